# baseline (speedup 1.0000x reference)
from math import sqrt

import numpy as np
import jax
import jax.numpy as jnp
from jax.experimental import pallas as pl
from jax.experimental.pallas import tpu as pltpu

_B, _L, _H, _D = 1, 2048, 16, 64
_U = 40
_NEG = -1.0e30


def _rotl(x, r):
    return ((x << np.uint32(r)) | (x >> np.uint32(32 - r))).astype(np.uint32)


def _threefry2x32(k1, k2, x0, x1):
    ks = [np.uint32(k1), np.uint32(k2),
          np.uint32(k1) ^ np.uint32(k2) ^ np.uint32(0x1BD11BDA)]
    x = [(x0 + ks[0]).astype(np.uint32), (x1 + ks[1]).astype(np.uint32)]
    ks_r = [ks[1], ks[2], ks[0]]
    rots = [(13, 15, 26, 6), (17, 29, 16, 24)]
    for i in range(5):
        for r in rots[0]:
            x[0] = (x[0] + x[1]).astype(np.uint32)
            x[1] = _rotl(x[1], r) ^ x[0]
        x = [(x[0] + ks_r[0]).astype(np.uint32),
             (x[1] + ks_r[1] + np.uint32(i + 1)).astype(np.uint32)]
        ks_r = ks_r[1:] + ks_r[:1]
        rots = rots[1:] + rots[:1]
    return x


def _sample_indices():
    b1, b2 = _threefry2x32(0, 42, np.zeros(2, np.uint32),
                           np.arange(2, dtype=np.uint32))
    n = _L * _U
    r1, r2 = _threefry2x32(b1[1], b2[1], np.zeros(n, np.uint32),
                           np.arange(n, dtype=np.uint32))
    return ((r1 ^ r2) % np.uint32(_L)).astype(np.int64).reshape(_L, _U)


def _count_matrix_t():
    idx = _sample_indices()
    c = np.zeros((_L, _L), dtype=np.float32)
    np.add.at(c, (np.arange(_L)[:, None], idx), 1.0)
    return c.T.copy()


_COUNT_T = _count_matrix_t()


def _measure_kernel(k_ref, q_ref, cnt_ref, m_ref):
    q = q_ref[0]
    k = k_ref[0]
    cnt = cnt_ref[...]
    s_t = jax.lax.dot_general(k, q, (((1,), (1,)), ((), ())),
                              preferred_element_type=jnp.float32)
    m_max = jnp.max(jnp.where(cnt > 0.0, s_t, _NEG), axis=0, keepdims=True)
    m_sum = jnp.sum(s_t * cnt, axis=0, keepdims=True)
    m_ref[0] = m_max - m_sum * (1.0 / _L)


def _topk_kernel(m_ref, idx_ref):
    m = m_ref[...]
    lanes = jax.lax.broadcasted_iota(jnp.int32, (_H, _L), 1)
    for i in range(_U):
        mv = jnp.max(m, axis=1, keepdims=True)
        idx = jnp.min(jnp.where(m == mv, lanes, _L), axis=1, keepdims=True)
        idx_ref[:, i, :] = idx
        m = jnp.where(lanes == idx, -3.0e38, m)


def _attn_kernel(q_ref, k_ref, v_ref, idx_ref, out_ref):
    q = q_ref[0]
    k = k_ref[0]
    v = v_ref[0]
    idx = idx_ref[0]

    lanes = jax.lax.broadcasted_iota(jnp.int32, (_U, _L), 1)
    onehot = (lanes == idx).astype(jnp.float32)

    qr = jnp.dot(onehot, q, preferred_element_type=jnp.float32,
                 precision=jax.lax.Precision.HIGHEST)
    s2 = jax.lax.dot_general(qr, k, (((1,), (1,)), ((), ())),
                             preferred_element_type=jnp.float32)
    s2 = s2 * (1.0 / sqrt(_D))
    s2m = jnp.max(s2, axis=1, keepdims=True)
    e = jnp.exp(s2 - s2m)
    attn = e / jnp.sum(e, axis=1, keepdims=True)
    upd = jnp.dot(attn, v, preferred_element_type=jnp.float32)

    mean = jnp.sum(v, axis=0, keepdims=True) * (1.0 / _L)
    u2 = jnp.concatenate([upd, jnp.ones((_U, 1), jnp.float32)], axis=1)
    g = jax.lax.dot_general(onehot, u2, (((0,), (0,)), ((), ())),
                            preferred_element_type=jnp.float32,
                            precision=jax.lax.Precision.HIGHEST)
    out_ref[0] = g[:, :_D] + mean * (1.0 - g[:, _D:_D + 1])


def kernel(queries, keys, values):
    cnt_t = jnp.asarray(_COUNT_T)
    q = jnp.transpose(queries, (0, 2, 1, 3)).reshape(_H, _L, _D)
    k = jnp.transpose(keys, (0, 2, 1, 3)).reshape(_H, _L, _D)
    v = jnp.transpose(values, (0, 2, 1, 3)).reshape(_H, _L, _D)

    m_all = pl.pallas_call(
        _measure_kernel,
        grid=(_H,),
        in_specs=[
            pl.BlockSpec((1, _L, _D), lambda h: (h, 0, 0)),
            pl.BlockSpec((1, _L, _D), lambda h: (h, 0, 0)),
            pl.BlockSpec((_L, _L), lambda h: (0, 0)),
        ],
        out_specs=pl.BlockSpec((1, 1, _L), lambda h: (h, 0, 0)),
        out_shape=jax.ShapeDtypeStruct((_H, 1, _L), jnp.float32),
    )(k, q, cnt_t)

    idxs = pl.pallas_call(
        _topk_kernel,
        out_shape=jax.ShapeDtypeStruct((_H, _U, 1), jnp.int32),
    )(m_all.reshape(_H, _L))

    out = pl.pallas_call(
        _attn_kernel,
        grid=(_H,),
        in_specs=[
            pl.BlockSpec((1, _L, _D), lambda h: (h, 0, 0)),
            pl.BlockSpec((1, _L, _D), lambda h: (h, 0, 0)),
            pl.BlockSpec((1, _L, _D), lambda h: (h, 0, 0)),
            pl.BlockSpec((1, _U, 1), lambda h: (h, 0, 0)),
        ],
        out_specs=pl.BlockSpec((1, _L, _D), lambda h: (h, 0, 0)),
        out_shape=jax.ShapeDtypeStruct((_H, _L, _D), jnp.float32),
    )(q, k, v, idxs)
    return out.reshape(_B, _H, _L, _D)


# speedup vs baseline: 1.0032x; 1.0032x over previous
from math import sqrt

import numpy as np
import jax
import jax.numpy as jnp
from jax.experimental import pallas as pl
from jax.experimental.pallas import tpu as pltpu

_B, _L, _H, _D = 1, 2048, 16, 64
_U = 40
_NEG = -1.0e30


def _rotl(x, r):
    return ((x << np.uint32(r)) | (x >> np.uint32(32 - r))).astype(np.uint32)


def _threefry2x32(k1, k2, x0, x1):
    ks = [np.uint32(k1), np.uint32(k2),
          np.uint32(k1) ^ np.uint32(k2) ^ np.uint32(0x1BD11BDA)]
    x = [(x0 + ks[0]).astype(np.uint32), (x1 + ks[1]).astype(np.uint32)]
    ks_r = [ks[1], ks[2], ks[0]]
    rots = [(13, 15, 26, 6), (17, 29, 16, 24)]
    for i in range(5):
        for r in rots[0]:
            x[0] = (x[0] + x[1]).astype(np.uint32)
            x[1] = _rotl(x[1], r) ^ x[0]
        x = [(x[0] + ks_r[0]).astype(np.uint32),
             (x[1] + ks_r[1] + np.uint32(i + 1)).astype(np.uint32)]
        ks_r = ks_r[1:] + ks_r[:1]
        rots = rots[1:] + rots[:1]
    return x


def _sample_indices():
    b1, b2 = _threefry2x32(0, 42, np.zeros(2, np.uint32),
                           np.arange(2, dtype=np.uint32))
    n = _L * _U
    r1, r2 = _threefry2x32(b1[1], b2[1], np.zeros(n, np.uint32),
                           np.arange(n, dtype=np.uint32))
    return ((r1 ^ r2) % np.uint32(_L)).astype(np.int64).reshape(_L, _U)


def _count_matrix_t():
    idx = _sample_indices()
    c = np.zeros((_L, _L), dtype=np.float32)
    np.add.at(c, (np.arange(_L)[:, None], idx), 1.0)
    return c.T.copy()


_COUNT_T = _count_matrix_t()


def _measure_kernel(kf_ref, qf_ref, cnt_ref, m_ref):
    cnt = cnt_ref[...]
    for j in (0, 1):
        q = qf_ref[:, j * _D:(j + 1) * _D]
        k = kf_ref[:, j * _D:(j + 1) * _D]
        s_t = jax.lax.dot_general(k, q, (((1,), (1,)), ((), ())),
                                  preferred_element_type=jnp.float32)
        m_max = jnp.max(jnp.where(cnt > 0.0, s_t, _NEG), axis=0, keepdims=True)
        m_sum = jnp.sum(s_t * cnt, axis=0, keepdims=True)
        m_ref[j, 0, :] = (m_max - m_sum * (1.0 / _L))[0]


def _topk_kernel(m_ref, idx_ref):
    m = m_ref[...]
    lanes = jax.lax.broadcasted_iota(jnp.int32, (_H, _L), 1)
    for i in range(_U):
        ai = jnp.argmax(m, axis=1, keepdims=True).astype(jnp.int32)
        idx_ref[:, i, :] = ai
        m = jnp.where(lanes == ai, -3.0e38, m)


def _attn_kernel(qf_ref, kf_ref, vf_ref, idx_ref, out_ref):
    lanes = jax.lax.broadcasted_iota(jnp.int32, (_U, _L), 1)
    for j in (0, 1):
        q = qf_ref[:, j * _D:(j + 1) * _D]
        k = kf_ref[:, j * _D:(j + 1) * _D]
        v = vf_ref[:, j * _D:(j + 1) * _D]
        idx = idx_ref[j]

        onehot = (lanes == idx).astype(jnp.float32)
        qr = jnp.dot(onehot, q, preferred_element_type=jnp.float32,
                     precision=jax.lax.Precision.HIGHEST)
        s2 = jax.lax.dot_general(qr, k, (((1,), (1,)), ((), ())),
                                 preferred_element_type=jnp.float32)
        s2 = s2 * (1.0 / sqrt(_D))
        s2m = jnp.max(s2, axis=1, keepdims=True)
        e = jnp.exp(s2 - s2m)
        attn = e / jnp.sum(e, axis=1, keepdims=True)
        upd = jnp.dot(attn, v, preferred_element_type=jnp.float32)

        mean = jnp.sum(v, axis=0, keepdims=True) * (1.0 / _L)
        u2 = jnp.concatenate([upd, jnp.ones((_U, 1), jnp.float32)], axis=1)
        g = jax.lax.dot_general(onehot, u2, (((0,), (0,)), ((), ())),
                                preferred_element_type=jnp.float32,
                                precision=jax.lax.Precision.HIGHEST)
        out_ref[j] = g[:, :_D] + mean * (1.0 - g[:, _D:_D + 1])


def kernel(queries, keys, values):
    cnt_t = jnp.asarray(_COUNT_T)
    qf = queries.reshape(_L, _H * _D)
    kf = keys.reshape(_L, _H * _D)
    vf = values.reshape(_L, _H * _D)

    m_all = pl.pallas_call(
        _measure_kernel,
        grid=(_H // 2,),
        in_specs=[
            pl.BlockSpec((_L, 2 * _D), lambda hp: (0, hp)),
            pl.BlockSpec((_L, 2 * _D), lambda hp: (0, hp)),
            pl.BlockSpec((_L, _L), lambda hp: (0, 0)),
        ],
        out_specs=pl.BlockSpec((2, 1, _L), lambda hp: (hp, 0, 0)),
        out_shape=jax.ShapeDtypeStruct((_H, 1, _L), jnp.float32),
    )(kf, qf, cnt_t)

    idxs = pl.pallas_call(
        _topk_kernel,
        out_shape=jax.ShapeDtypeStruct((_H, _U, 1), jnp.int32),
    )(m_all.reshape(_H, _L))

    out = pl.pallas_call(
        _attn_kernel,
        grid=(_H // 2,),
        in_specs=[
            pl.BlockSpec((_L, 2 * _D), lambda hp: (0, hp)),
            pl.BlockSpec((_L, 2 * _D), lambda hp: (0, hp)),
            pl.BlockSpec((_L, 2 * _D), lambda hp: (0, hp)),
            pl.BlockSpec((2, _U, 1), lambda hp: (hp, 0, 0)),
        ],
        out_specs=pl.BlockSpec((2, _L, _D), lambda hp: (hp, 0, 0)),
        out_shape=jax.ShapeDtypeStruct((_H, _L, _D), jnp.float32),
    )(qf, kf, vf, idxs)
    return out.reshape(_B, _H, _L, _D)


# speedup vs baseline: 1.0268x; 1.0236x over previous
from math import sqrt

import numpy as np
import jax
import jax.numpy as jnp
from jax.experimental import pallas as pl
from jax.experimental.pallas import tpu as pltpu

_B, _L, _H, _D = 1, 2048, 16, 64
_U = 40
_NEG = -1.0e30


def _rotl(x, r):
    return ((x << np.uint32(r)) | (x >> np.uint32(32 - r))).astype(np.uint32)


def _threefry2x32(k1, k2, x0, x1):
    ks = [np.uint32(k1), np.uint32(k2),
          np.uint32(k1) ^ np.uint32(k2) ^ np.uint32(0x1BD11BDA)]
    x = [(x0 + ks[0]).astype(np.uint32), (x1 + ks[1]).astype(np.uint32)]
    ks_r = [ks[1], ks[2], ks[0]]
    rots = [(13, 15, 26, 6), (17, 29, 16, 24)]
    for i in range(5):
        for r in rots[0]:
            x[0] = (x[0] + x[1]).astype(np.uint32)
            x[1] = _rotl(x[1], r) ^ x[0]
        x = [(x[0] + ks_r[0]).astype(np.uint32),
             (x[1] + ks_r[1] + np.uint32(i + 1)).astype(np.uint32)]
        ks_r = ks_r[1:] + ks_r[:1]
        rots = rots[1:] + rots[:1]
    return x


def _sample_indices():
    b1, b2 = _threefry2x32(0, 42, np.zeros(2, np.uint32),
                           np.arange(2, dtype=np.uint32))
    n = _L * _U
    r1, r2 = _threefry2x32(b1[1], b2[1], np.zeros(n, np.uint32),
                           np.arange(n, dtype=np.uint32))
    return ((r1 ^ r2) % np.uint32(_L)).astype(np.int64).reshape(_L, _U)


def _count_matrix_t():
    idx = _sample_indices()
    c = np.zeros((_L, _L), dtype=np.float32)
    np.add.at(c, (np.arange(_L)[:, None], idx), 1.0)
    return c.T.copy()


_COUNT_T = _count_matrix_t()


def _fused_kernel(qf_ref, kf_ref, vf_ref, cnt_ref, out_ref, m_s, idx_s):
    phase = pl.program_id(0)
    hp = pl.program_id(1)

    @pl.when(phase == 0)
    def _measure():
        cnt = cnt_ref[...]
        for j in (0, 1):
            q = qf_ref[:, j * _D:(j + 1) * _D]
            k = kf_ref[:, j * _D:(j + 1) * _D]
            s_t = jax.lax.dot_general(k, q, (((1,), (1,)), ((), ())),
                                      preferred_element_type=jnp.float32)
            m_max = jnp.max(jnp.where(cnt > 0.0, s_t, _NEG), axis=0,
                            keepdims=True)
            m_sum = jnp.sum(s_t * cnt, axis=0, keepdims=True)
            m_s[pl.ds(2 * hp + j, 1), :] = m_max - m_sum * (1.0 / _L)

    @pl.when((phase == 1) & (hp == 0))
    def _topk():
        m = m_s[...]
        lanes = jax.lax.broadcasted_iota(jnp.int32, (_H, _L), 1)
        for i in range(_U):
            ai = jnp.argmax(m, axis=1, keepdims=True).astype(jnp.int32)
            idx_s[:, i:i + 1] = ai
            m = jnp.where(lanes == ai, -3.0e38, m)

    @pl.when(phase == 1)
    def _attend():
        lanes = jax.lax.broadcasted_iota(jnp.int32, (_U, _L), 1)
        for j in (0, 1):
            q = qf_ref[:, j * _D:(j + 1) * _D]
            k = kf_ref[:, j * _D:(j + 1) * _D]
            v = vf_ref[:, j * _D:(j + 1) * _D]
            row = idx_s[pl.ds(2 * hp + j, 1), 0:_U]
            idx = jnp.transpose(row, (1, 0))

            onehot = (lanes == idx).astype(jnp.float32)
            qr = jnp.dot(onehot, q, preferred_element_type=jnp.float32,
                         precision=jax.lax.Precision.HIGHEST)
            s2 = jax.lax.dot_general(qr, k, (((1,), (1,)), ((), ())),
                                     preferred_element_type=jnp.float32)
            s2 = s2 * (1.0 / sqrt(_D))
            s2m = jnp.max(s2, axis=1, keepdims=True)
            e = jnp.exp(s2 - s2m)
            attn = e / jnp.sum(e, axis=1, keepdims=True)
            upd = jnp.dot(attn, v, preferred_element_type=jnp.float32)

            mean = jnp.sum(v, axis=0, keepdims=True) * (1.0 / _L)
            u2 = jnp.concatenate([upd, jnp.ones((_U, 1), jnp.float32)],
                                 axis=1)
            g = jax.lax.dot_general(onehot, u2, (((0,), (0,)), ((), ())),
                                    preferred_element_type=jnp.float32,
                                    precision=jax.lax.Precision.HIGHEST)
            out_ref[j] = g[:, :_D] + mean * (1.0 - g[:, _D:_D + 1])


def kernel(queries, keys, values):
    cnt_t = jnp.asarray(_COUNT_T)
    qf = queries.reshape(_L, _H * _D)
    kf = keys.reshape(_L, _H * _D)
    vf = values.reshape(_L, _H * _D)

    out = pl.pallas_call(
        _fused_kernel,
        grid=(2, _H // 2),
        in_specs=[
            pl.BlockSpec((_L, 2 * _D), lambda ph, hp: (0, hp)),
            pl.BlockSpec((_L, 2 * _D), lambda ph, hp: (0, hp)),
            pl.BlockSpec((_L, 2 * _D), lambda ph, hp: (0, hp)),
            pl.BlockSpec((_L, _L), lambda ph, hp: (0, 0)),
        ],
        out_specs=pl.BlockSpec((2, _L, _D), lambda ph, hp: (hp, 0, 0)),
        out_shape=jax.ShapeDtypeStruct((_H, _L, _D), jnp.float32),
        scratch_shapes=[
            pltpu.VMEM((_H, _L), jnp.float32),
            pltpu.VMEM((_H, 64), jnp.int32),
        ],
    )(qf, kf, vf, cnt_t)
    return out.reshape(_B, _H, _L, _D)


# speedup vs baseline: 1.0582x; 1.0305x over previous
from math import sqrt

import numpy as np
import jax
import jax.numpy as jnp
from jax.experimental import pallas as pl
from jax.experimental.pallas import tpu as pltpu

_B, _L, _H, _D = 1, 2048, 16, 64
_U = 40
_NEG = -1.0e30


def _rotl(x, r):
    return ((x << np.uint32(r)) | (x >> np.uint32(32 - r))).astype(np.uint32)


def _threefry2x32(k1, k2, x0, x1):
    ks = [np.uint32(k1), np.uint32(k2),
          np.uint32(k1) ^ np.uint32(k2) ^ np.uint32(0x1BD11BDA)]
    x = [(x0 + ks[0]).astype(np.uint32), (x1 + ks[1]).astype(np.uint32)]
    ks_r = [ks[1], ks[2], ks[0]]
    rots = [(13, 15, 26, 6), (17, 29, 16, 24)]
    for i in range(5):
        for r in rots[0]:
            x[0] = (x[0] + x[1]).astype(np.uint32)
            x[1] = _rotl(x[1], r) ^ x[0]
        x = [(x[0] + ks_r[0]).astype(np.uint32),
             (x[1] + ks_r[1] + np.uint32(i + 1)).astype(np.uint32)]
        ks_r = ks_r[1:] + ks_r[:1]
        rots = rots[1:] + rots[:1]
    return x


def _sample_indices():
    b1, b2 = _threefry2x32(0, 42, np.zeros(2, np.uint32),
                           np.arange(2, dtype=np.uint32))
    n = _L * _U
    r1, r2 = _threefry2x32(b1[1], b2[1], np.zeros(n, np.uint32),
                           np.arange(n, dtype=np.uint32))
    return ((r1 ^ r2) % np.uint32(_L)).astype(np.int64).reshape(_L, _U)


def _count_matrix_t():
    idx = _sample_indices()
    c = np.zeros((_L, _L), dtype=np.float32)
    np.add.at(c, (np.arange(_L)[:, None], idx), 1.0)
    return c.T.copy()


_COUNT_T = _count_matrix_t()
_BIAS_T = np.where(_COUNT_T > 0.0, 0.0, np.float32(_NEG)).astype(np.float32)

_KC = 512


def _fused_kernel(qf_ref, kf_ref, vf_ref, cnt_ref, bias_ref, out_ref,
                  m_s, idx_s):
    phase = pl.program_id(0)
    hp = pl.program_id(1)

    @pl.when(phase == 0)
    def _measure():
        for j in (0, 1):
            q = qf_ref[:, j * _D:(j + 1) * _D]
            m_max = None
            m_sum = None
            for c in range(_L // _KC):
                kc = kf_ref[c * _KC:(c + 1) * _KC, j * _D:(j + 1) * _D]
                s_c = jax.lax.dot_general(kc, q, (((1,), (1,)), ((), ())),
                                          preferred_element_type=jnp.float32)
                bias_c = bias_ref[c * _KC:(c + 1) * _KC, :]
                cnt_c = cnt_ref[c * _KC:(c + 1) * _KC, :]
                mx = jnp.max(s_c + bias_c, axis=0, keepdims=True)
                sm = jnp.sum(s_c * cnt_c, axis=0, keepdims=True)
                m_max = mx if m_max is None else jnp.maximum(m_max, mx)
                m_sum = sm if m_sum is None else m_sum + sm
            m_s[pl.ds(2 * hp + j, 1), :] = m_max - m_sum * (1.0 / _L)

    @pl.when((phase == 1) & (hp == 0))
    def _topk():
        m = m_s[...]
        lanes = jax.lax.broadcasted_iota(jnp.int32, (_H, _L), 1)
        for i in range(_U):
            ai = jnp.argmax(m, axis=1, keepdims=True).astype(jnp.int32)
            idx_s[:, i:i + 1] = ai
            m = jnp.where(lanes == ai, -3.0e38, m)

    @pl.when(phase == 1)
    def _attend():
        lanes = jax.lax.broadcasted_iota(jnp.int32, (_U, _L), 1)
        for j in (0, 1):
            q = qf_ref[:, j * _D:(j + 1) * _D]
            k = kf_ref[:, j * _D:(j + 1) * _D]
            v = vf_ref[:, j * _D:(j + 1) * _D]
            row = idx_s[pl.ds(2 * hp + j, 1), 0:_U]
            idx = jnp.transpose(row, (1, 0))

            onehot = (lanes == idx).astype(jnp.float32)
            qr = jnp.dot(onehot, q, preferred_element_type=jnp.float32,
                         precision=jax.lax.Precision.HIGHEST)
            s2 = jax.lax.dot_general(qr, k, (((1,), (1,)), ((), ())),
                                     preferred_element_type=jnp.float32)
            s2 = s2 * (1.0 / sqrt(_D))
            s2m = jnp.max(s2, axis=1, keepdims=True)
            e = jnp.exp(s2 - s2m)
            attn = e / jnp.sum(e, axis=1, keepdims=True)
            upd = jnp.dot(attn, v, preferred_element_type=jnp.float32)

            mean = jnp.sum(v, axis=0, keepdims=True) * (1.0 / _L)
            u2 = jnp.concatenate([upd, jnp.ones((_U, 1), jnp.float32)],
                                 axis=1)
            g = jax.lax.dot_general(onehot, u2, (((0,), (0,)), ((), ())),
                                    preferred_element_type=jnp.float32,
                                    precision=jax.lax.Precision.HIGHEST)
            out_ref[j] = g[:, :_D] + mean * (1.0 - g[:, _D:_D + 1])


def kernel(queries, keys, values):
    cnt_t = jnp.asarray(_COUNT_T)
    qf = queries.reshape(_L, _H * _D)
    kf = keys.reshape(_L, _H * _D)
    vf = values.reshape(_L, _H * _D)

    out = pl.pallas_call(
        _fused_kernel,
        grid=(2, _H // 2),
        in_specs=[
            pl.BlockSpec((_L, 2 * _D), lambda ph, hp: (0, hp)),
            pl.BlockSpec((_L, 2 * _D), lambda ph, hp: (0, hp)),
            pl.BlockSpec((_L, 2 * _D), lambda ph, hp: (0, hp)),
            pl.BlockSpec((_L, _L), lambda ph, hp: (0, 0)),
            pl.BlockSpec((_L, _L), lambda ph, hp: (0, 0)),
        ],
        out_specs=pl.BlockSpec((2, _L, _D), lambda ph, hp: (hp, 0, 0)),
        out_shape=jax.ShapeDtypeStruct((_H, _L, _D), jnp.float32),
        scratch_shapes=[
            pltpu.VMEM((_H, _L), jnp.float32),
            pltpu.VMEM((_H, 64), jnp.int32),
        ],
    )(qf, kf, vf, cnt_t, jnp.asarray(_BIAS_T))
    return out.reshape(_B, _H, _L, _D)


# speedup vs baseline: 1.1181x; 1.0566x over previous
from math import sqrt

import numpy as np
import jax
import jax.numpy as jnp
from jax.experimental import pallas as pl
from jax.experimental.pallas import tpu as pltpu

_B, _L, _H, _D = 1, 2048, 16, 64
_U = 40
_NEG = -1.0e30


def _rotl(x, r):
    return ((x << np.uint32(r)) | (x >> np.uint32(32 - r))).astype(np.uint32)


def _threefry2x32(k1, k2, x0, x1):
    ks = [np.uint32(k1), np.uint32(k2),
          np.uint32(k1) ^ np.uint32(k2) ^ np.uint32(0x1BD11BDA)]
    x = [(x0 + ks[0]).astype(np.uint32), (x1 + ks[1]).astype(np.uint32)]
    ks_r = [ks[1], ks[2], ks[0]]
    rots = [(13, 15, 26, 6), (17, 29, 16, 24)]
    for i in range(5):
        for r in rots[0]:
            x[0] = (x[0] + x[1]).astype(np.uint32)
            x[1] = _rotl(x[1], r) ^ x[0]
        x = [(x[0] + ks_r[0]).astype(np.uint32),
             (x[1] + ks_r[1] + np.uint32(i + 1)).astype(np.uint32)]
        ks_r = ks_r[1:] + ks_r[:1]
        rots = rots[1:] + rots[:1]
    return x


def _sample_indices():
    b1, b2 = _threefry2x32(0, 42, np.zeros(2, np.uint32),
                           np.arange(2, dtype=np.uint32))
    n = _L * _U
    r1, r2 = _threefry2x32(b1[1], b2[1], np.zeros(n, np.uint32),
                           np.arange(n, dtype=np.uint32))
    return ((r1 ^ r2) % np.uint32(_L)).astype(np.int64).reshape(_L, _U)


def _count_matrix_t():
    idx = _sample_indices()
    c = np.zeros((_L, _L), dtype=np.float32)
    np.add.at(c, (np.arange(_L)[:, None], idx), 1.0)
    return c.T.copy()


_COUNT_T = _count_matrix_t()
_BIAS_T = np.where(_COUNT_T > 0.0, 0.0, np.float32(_NEG)).astype(np.float32)

_KC = 512


def _fused_kernel(qf_ref, kf_ref, vf_ref, cnt_ref, bias_ref, out_ref,
                  m_s, idx_s):
    phase = pl.program_id(0)
    hp = pl.program_id(1)

    @pl.when(phase == 0)
    def _measure():
        for j in (0, 1):
            q = qf_ref[:, j * _D:(j + 1) * _D]
            m_max = None
            m_sum = None
            for c in range(_L // _KC):
                kc = kf_ref[c * _KC:(c + 1) * _KC, j * _D:(j + 1) * _D]
                s_c = jax.lax.dot_general(kc, q, (((1,), (1,)), ((), ())),
                                          preferred_element_type=jnp.float32)
                bias_c = bias_ref[c * _KC:(c + 1) * _KC, :]
                cnt_c = cnt_ref[c * _KC:(c + 1) * _KC, :]
                mx = jnp.max(s_c + bias_c, axis=0, keepdims=True)
                sm = jnp.sum(s_c * cnt_c, axis=0, keepdims=True)
                m_max = mx if m_max is None else jnp.maximum(m_max, mx)
                m_sum = sm if m_sum is None else m_sum + sm
            m_s[pl.ds(2 * hp + j, 1), :] = m_max - m_sum * (1.0 / _L)

    @pl.when((phase == 1) & (hp == 0))
    def _topk():
        m = m_s[...]
        lanes = jax.lax.broadcasted_iota(jnp.int32, (_H, _L), 1)
        for i in range(_U):
            ai = jnp.argmax(m, axis=1, keepdims=True).astype(jnp.int32)
            idx_s[:, i:i + 1] = ai
            m = jnp.where(lanes == ai, -3.0e38, m)

    @pl.when(phase == 1)
    def _attend():
        lanes = jax.lax.broadcasted_iota(jnp.int32, (_U, _L), 1)
        for j in (0, 1):
            q = qf_ref[:, j * _D:(j + 1) * _D]
            k = kf_ref[:, j * _D:(j + 1) * _D]
            v = vf_ref[:, j * _D:(j + 1) * _D]
            row = idx_s[pl.ds(2 * hp + j, 1), 0:_U]
            idx = jnp.transpose(row, (1, 0))

            onehot = (lanes == idx).astype(jnp.float32)
            qr = jnp.dot(onehot, q, preferred_element_type=jnp.float32)
            s2 = jax.lax.dot_general(qr, k, (((1,), (1,)), ((), ())),
                                     preferred_element_type=jnp.float32)
            s2 = s2 * (1.0 / sqrt(_D))
            s2m = jnp.max(s2, axis=1, keepdims=True)
            e = jnp.exp(s2 - s2m)
            attn = e / jnp.sum(e, axis=1, keepdims=True)
            upd = jnp.dot(attn, v, preferred_element_type=jnp.float32)

            mean = jnp.sum(v, axis=0, keepdims=True) * (1.0 / _L)
            u2 = jnp.concatenate([upd, jnp.ones((_U, 1), jnp.float32)],
                                 axis=1)
            g = jax.lax.dot_general(onehot, u2, (((0,), (0,)), ((), ())),
                                    preferred_element_type=jnp.float32,
                                    precision=jax.lax.Precision.HIGHEST)
            out_ref[j] = g[:, :_D] + mean * (1.0 - g[:, _D:_D + 1])


def kernel(queries, keys, values):
    cnt_t = jnp.asarray(_COUNT_T)
    qf = queries.reshape(_L, _H * _D)
    kf = keys.reshape(_L, _H * _D)
    vf = values.reshape(_L, _H * _D)

    out = pl.pallas_call(
        _fused_kernel,
        grid=(2, _H // 2),
        in_specs=[
            pl.BlockSpec((_L, 2 * _D), lambda ph, hp: (0, hp)),
            pl.BlockSpec((_L, 2 * _D), lambda ph, hp: (0, hp)),
            pl.BlockSpec((_L, 2 * _D), lambda ph, hp: (0, hp)),
            pl.BlockSpec((_L, _L), lambda ph, hp: (0, 0)),
            pl.BlockSpec((_L, _L), lambda ph, hp: (0, 0)),
        ],
        out_specs=pl.BlockSpec((2, _L, _D), lambda ph, hp: (hp, 0, 0)),
        out_shape=jax.ShapeDtypeStruct((_H, _L, _D), jnp.float32),
        scratch_shapes=[
            pltpu.VMEM((_H, _L), jnp.float32),
            pltpu.VMEM((_H, 64), jnp.int32),
        ],
    )(qf, kf, vf, cnt_t, jnp.asarray(_BIAS_T))
    return out.reshape(_B, _H, _L, _D)
